# baseline (speedup 1.0000x reference)
import functools

import jax
import jax.numpy as jnp
from jax import lax
from jax.experimental import pallas as pl
from jax.experimental.pallas import tpu as pltpu

NUM_USERS = 25000
NUM_ITEMS = 25000
N_NODES = NUM_USERS + NUM_ITEMS
N_EDGES = 800000
LATENT_DIM = 64
N_LAYERS = 3
GROUPS = 64
BATCH = 1024

_BN = 512


def _rating_body(users_kernel_norm_ref, anchor_emb_ref, items_ref, out_ref):
    users_emb = jnp.dot(users_kernel_norm_ref[...], anchor_emb_ref[...],
                        preferred_element_type=jnp.float32)
    logits = lax.dot_general(users_emb, items_ref[...],
                             (((1,), (1,)), ((), ())),
                             preferred_element_type=jnp.float32)
    out_ref[...] = jax.nn.sigmoid(logits)


def _rating_matmul(users_kernel_norm, anchor_emb, all_items):
    n_blocks = pl.cdiv(NUM_ITEMS, _BN)
    return pl.pallas_call(
        _rating_body,
        grid=(n_blocks,),
        in_specs=[
            pl.BlockSpec((BATCH, GROUPS), lambda i: (0, 0)),
            pl.BlockSpec((GROUPS, LATENT_DIM), lambda i: (0, 0)),
            pl.BlockSpec((_BN, LATENT_DIM), lambda i: (i, 0)),
        ],
        out_specs=pl.BlockSpec((BATCH, _BN), lambda i: (0, i)),
        out_shape=jax.ShapeDtypeStruct((BATCH, NUM_ITEMS), jnp.float32),
    )(users_kernel_norm, anchor_emb, all_items)


def kernel(embedding_user, embedding_item, edge_index, edge_weight, train_kernel, anchors, users):
    all_emb = jnp.concatenate([embedding_user, embedding_item], axis=0)
    dst = edge_index[0]
    src = edge_index[1]
    acc = all_emb
    for _ in range(N_LAYERS):
        msgs = edge_weight[:, None] * jnp.take(all_emb, src, axis=0)
        all_emb = jax.ops.segment_sum(msgs, dst, num_segments=N_NODES)
        acc = acc + all_emb
    light_out = acc * (1.0 / (N_LAYERS + 1))
    all_items = light_out[NUM_USERS:]

    anchor_emb = jnp.take(light_out[:NUM_USERS], anchors, axis=0)
    users_kernel = jnp.take(train_kernel, users, axis=0)
    users_kernel_norm = users_kernel / jnp.sum(users_kernel, axis=1, keepdims=True)

    return _rating_matmul(users_kernel_norm, anchor_emb, all_items)


# speedup vs baseline: 8.3834x; 8.3834x over previous
import functools

import jax
import jax.numpy as jnp
from jax import lax
from jax.experimental import pallas as pl
from jax.experimental.pallas import tpu as pltpu
from jax.experimental.pallas import tpu_sc as plsc

NUM_USERS = 25000
NUM_ITEMS = 25000
N_NODES = NUM_USERS + NUM_ITEMS
N_EDGES = 800000
LATENT_DIM = 64
N_LAYERS = 3
GROUPS = 64
BATCH = 1024

NC = 2
NS = 16
HALF = LATENT_DIM // NC
CH = 128
SEG = 28
NSEG = 14
EPT = SEG * NSEG * CH
E_PAD = EPT * NS
N_PAD = 50048
ROWS_PT = N_PAD // NS
RCH = 64
NRCH = ROWS_PT // RCH
RTAIL = ROWS_PT - NRCH * RCH

_BN = 512


def _scale_rows(rows, wseg, j):
    for g in range(CH // 16):
        wv = wseg[j, pl.ds(g * 16, 16)]
        for e in range(16):
            s = wv[e]
            r = g * 16 + e
            rows[r, 0:16] = rows[r, 0:16] * s
            rows[r, 16:32] = rows[r, 16:32] * s


def _sc_body(emb0, srcs, dsts, ws, acc_out, stage0, stage1, stage2,
             idx_b, dst_b, w_b, rows0, rows1, av, bv, cv, tmp_v, zeros_v,
             acc_sp, semA, semB):
    cid = lax.axis_index("c")
    sid = lax.axis_index("s")
    st0 = stage0.at[cid]
    st1 = stage1.at[cid]
    st2 = stage2.at[cid]
    acch = acc_out.at[cid]

    zf = jnp.zeros((16,), jnp.float32)

    @pl.loop(0, RCH)
    def _(r):
        zeros_v[r, 0:16] = zf
        zeros_v[r, 16:32] = zf

    def wait_a():
        pltpu.make_async_copy(st0.at[pl.ds(0, CH)], rows0, semA).wait()

    def edge_phase(table):
        @pl.loop(0, NSEG)
        def _(t):
            base = t * SEG
            pltpu.sync_copy(srcs.at[sid, pl.ds(base, SEG)], idx_b)
            pltpu.sync_copy(dsts.at[sid, pl.ds(base, SEG)], dst_b)
            pltpu.sync_copy(ws.at[sid, pl.ds(base, SEG)], w_b)
            pltpu.async_copy(table.at[idx_b.at[0]], rows0, semA)

            @pl.loop(0, SEG // 2)
            def _(m):
                j0 = 2 * m
                d1 = pltpu.async_copy(table.at[idx_b.at[j0 + 1]], rows1, semB)
                wait_a()
                _scale_rows(rows0, w_b, j0)
                pltpu.sync_copy(rows0, acc_sp.at[dst_b.at[j0]], add=True)

                @pl.when(j0 + 2 < SEG)
                def _():
                    pltpu.async_copy(table.at[idx_b.at[j0 + 2]], rows0, semA)

                d1.wait()
                _scale_rows(rows1, w_b, j0 + 1)
                pltpu.sync_copy(rows1, acc_sp.at[dst_b.at[j0 + 1]], add=True)

    def stage_chunk(stage_ref, rbase, n):
        pltpu.sync_copy(acc_sp.at[pl.ds(rbase, n)], tmp_v.at[pl.ds(0, n)])
        pltpu.sync_copy(tmp_v.at[pl.ds(0, n)], stage_ref.at[pl.ds(rbase, n)])
        pltpu.sync_copy(zeros_v.at[pl.ds(0, n)], acc_sp.at[pl.ds(rbase, n)])

    def stage_and_zero(stage_ref):
        @pl.loop(0, NRCH)
        def _(k):
            stage_chunk(stage_ref, sid * ROWS_PT + k * RCH, RCH)
        stage_chunk(stage_ref, sid * ROWS_PT + NRCH * RCH, RTAIL)

    def final_chunk(rbase, n):
        pltpu.sync_copy(acc_sp.at[pl.ds(rbase, n)], tmp_v.at[pl.ds(0, n)])
        pltpu.sync_copy(st0.at[pl.ds(rbase, n)], av.at[pl.ds(0, n)])
        pltpu.sync_copy(st1.at[pl.ds(rbase, n)], bv.at[pl.ds(0, n)])
        pltpu.sync_copy(st2.at[pl.ds(rbase, n)], cv.at[pl.ds(0, n)])

        @pl.loop(0, n, unroll=4)
        def _(r):
            av[r, 0:16] = av[r, 0:16] + bv[r, 0:16] + cv[r, 0:16] + tmp_v[r, 0:16]
            av[r, 16:32] = av[r, 16:32] + bv[r, 16:32] + cv[r, 16:32] + tmp_v[r, 16:32]

        pltpu.sync_copy(av.at[pl.ds(0, n)], acch.at[pl.ds(rbase, n)])

    def final_sum():
        @pl.loop(0, NRCH)
        def _(k):
            final_chunk(sid * ROWS_PT + k * RCH, RCH)
        final_chunk(sid * ROWS_PT + NRCH * RCH, RTAIL)

    def zero_chunk(rbase, n):
        pltpu.sync_copy(zeros_v.at[pl.ds(0, n)], acc_sp.at[pl.ds(rbase, n)])

    def split_chunk(rbase, n):
        pltpu.sync_copy(emb0.at[pl.ds(rbase, n), pl.ds(cid * HALF, HALF)],
                        av.at[pl.ds(0, n)])
        pltpu.sync_copy(av.at[pl.ds(0, n)], st0.at[pl.ds(rbase, n)])

    @pl.loop(0, NRCH)
    def _(k):
        zero_chunk(sid * ROWS_PT + k * RCH, RCH)
        split_chunk(sid * ROWS_PT + k * RCH, RCH)
    zero_chunk(sid * ROWS_PT + NRCH * RCH, RTAIL)
    split_chunk(sid * ROWS_PT + NRCH * RCH, RTAIL)

    plsc.subcore_barrier()
    edge_phase(st0)
    plsc.subcore_barrier()
    stage_and_zero(st1)
    plsc.subcore_barrier()
    edge_phase(st1)
    plsc.subcore_barrier()
    stage_and_zero(st2)
    plsc.subcore_barrier()
    edge_phase(st2)
    plsc.subcore_barrier()
    final_sum()


def _sc_propagate(emb0, srcs, dsts, ws):
    mesh = plsc.VectorSubcoreMesh(core_axis_name="c", subcore_axis_name="s",
                                  num_cores=NC, num_subcores=NS)
    f = pl.kernel(
        _sc_body,
        out_type=(
            jax.ShapeDtypeStruct((NC, N_PAD, HALF), jnp.float32),
            jax.ShapeDtypeStruct((NC, N_PAD, HALF), jnp.float32),
            jax.ShapeDtypeStruct((NC, N_PAD, HALF), jnp.float32),
            jax.ShapeDtypeStruct((NC, N_PAD, HALF), jnp.float32),
        ),
        mesh=mesh,
        scratch_types=[
            pltpu.VMEM((SEG, CH), jnp.int32),
            pltpu.VMEM((SEG, CH), jnp.int32),
            pltpu.VMEM((SEG, CH), jnp.float32),
            pltpu.VMEM((CH, HALF), jnp.float32),
            pltpu.VMEM((CH, HALF), jnp.float32),
            pltpu.VMEM((RCH, HALF), jnp.float32),
            pltpu.VMEM((RCH, HALF), jnp.float32),
            pltpu.VMEM((RCH, HALF), jnp.float32),
            pltpu.VMEM((RCH, HALF), jnp.float32),
            pltpu.VMEM((RCH, HALF), jnp.float32),
            pltpu.VMEM_SHARED((N_PAD, HALF), jnp.float32),
            pltpu.SemaphoreType.DMA,
            pltpu.SemaphoreType.DMA,
        ],
        compiler_params=pltpu.CompilerParams(use_tc_tiling_on_sc=False),
    )
    return f(emb0, srcs, dsts, ws)


def _rating_body(users_kernel_norm_ref, anchor_sum_ref, items0_ref, items1_ref,
                 out_ref):
    users_emb = jnp.dot(users_kernel_norm_ref[...], anchor_sum_ref[...],
                        preferred_element_type=jnp.float32) * 0.0625
    logits = lax.dot_general(users_emb[:, :HALF], items0_ref[...],
                             (((1,), (1,)), ((), ())),
                             preferred_element_type=jnp.float32)
    logits += lax.dot_general(users_emb[:, HALF:], items1_ref[...],
                              (((1,), (1,)), ((), ())),
                              preferred_element_type=jnp.float32)
    out_ref[...] = jax.nn.sigmoid(logits)


def _rating_matmul(users_kernel_norm, anchor_sum, items0, items1):
    n_blocks = pl.cdiv(NUM_ITEMS, _BN)
    return pl.pallas_call(
        _rating_body,
        grid=(n_blocks,),
        in_specs=[
            pl.BlockSpec((BATCH, GROUPS), lambda i: (0, 0)),
            pl.BlockSpec((GROUPS, LATENT_DIM), lambda i: (0, 0)),
            pl.BlockSpec((_BN, HALF), lambda i: (i, 0)),
            pl.BlockSpec((_BN, HALF), lambda i: (i, 0)),
        ],
        out_specs=pl.BlockSpec((BATCH, _BN), lambda i: (0, i)),
        out_shape=jax.ShapeDtypeStruct((BATCH, NUM_ITEMS), jnp.float32),
    )(users_kernel_norm, anchor_sum, items0, items1)


def kernel(embedding_user, embedding_item, edge_index, edge_weight, train_kernel, anchors, users):
    all_emb = jnp.concatenate([embedding_user, embedding_item], axis=0)
    emb0 = jnp.pad(all_emb, ((0, N_PAD - N_NODES), (0, 0)))

    dst = edge_index[0].astype(jnp.int32)
    src = edge_index[1].astype(jnp.int32)
    w = edge_weight.astype(jnp.float32)
    npad = E_PAD - N_EDGES
    pad_idx = (jnp.arange(npad, dtype=jnp.int32) * 16) % N_NODES
    srcs = jnp.concatenate([src, pad_idx]).reshape(NS, SEG * NSEG, CH)
    dsts = jnp.concatenate([dst, pad_idx]).reshape(NS, SEG * NSEG, CH)
    ws = jnp.concatenate([w, jnp.zeros((npad,), jnp.float32)]).reshape(NS, SEG * NSEG, CH)
    acc2, _s0, _s1, _s2 = _sc_propagate(emb0, srcs, dsts, ws)
    h0 = acc2[0, :N_NODES]
    h1 = acc2[1, :N_NODES]

    anchor_sum = jnp.concatenate(
        [jnp.take(h0[:NUM_USERS], anchors, axis=0),
         jnp.take(h1[:NUM_USERS], anchors, axis=0)], axis=1)
    items0 = h0[NUM_USERS:]
    items1 = h1[NUM_USERS:]
    users_kernel = jnp.take(train_kernel, users, axis=0)
    users_kernel_norm = users_kernel / jnp.sum(users_kernel, axis=1, keepdims=True)

    return _rating_matmul(users_kernel_norm, anchor_sum, items0, items1)
